# Initial kernel scaffold; baseline (speedup 1.0000x reference)
import jax
import jax.numpy as jnp
from jax.experimental import pallas as pl


def kernel(coeffs, edge_index, W0, b0, a0, W1, a1, W2, a2, W3, b3):
    raise NotImplementedError("write your pallas kernel here")


# speedup vs baseline: 38.8496x; 38.8496x over previous
import functools

import jax
import jax.numpy as jnp
from jax import lax
from jax.experimental import pallas as pl
from jax.experimental.pallas import tpu as pltpu
from jax.experimental.pallas import tpu_sc as plsc

N = 10000
E = 320000
D = 128

NC = 2
NS = 16
NW = NC * NS
EPW = E // NW
LANES = 16
STEPS = EPW // LANES


def _mlp_body(x_ref, w0_ref, b0_ref, a0_ref, w1_ref, a1_ref, w2_ref, a2_ref,
              w3t_ref, b3_ref, out_ref):
    x = x_ref[...]
    dn = (((1,), (1,)), ((), ()))
    h = lax.dot_general(x, w0_ref[...], dn, preferred_element_type=jnp.float32)
    h = h + b0_ref[...]
    h = jnp.where(h >= 0, h, a0_ref[0, 0] * h)
    h = lax.dot_general(h, w1_ref[...], dn, preferred_element_type=jnp.float32)
    h = jnp.where(h >= 0, h, a1_ref[0, 0] * h)
    h = lax.dot_general(h, w2_ref[...], dn, preferred_element_type=jnp.float32)
    h = jnp.where(h >= 0, h, a2_ref[0, 0] * h)
    ang = lax.dot_general(h, w3t_ref[...], (((1,), (0,)), ((), ())),
                          preferred_element_type=jnp.float32)
    out_ref[...] = 0.25 * (ang + b3_ref[0, 0])


def _mlp(coeffs, W0, b0, a0, W1, a1, W2, a2, W3, b3):
    bn = 1000
    grid = (N // bn,)
    full = lambda shape: pl.BlockSpec(shape, lambda i: (0, 0))
    return pl.pallas_call(
        _mlp_body,
        grid=grid,
        in_specs=[
            pl.BlockSpec((bn, D), lambda i: (i, 0)),
            full((D, D)), full((1, D)), full((1, 1)),
            full((D, D)), full((1, 1)),
            full((D, D)), full((1, 1)),
            full((D, 1)), full((1, 1)),
        ],
        out_specs=pl.BlockSpec((bn, 1), lambda i: (i, 0)),
        out_shape=jax.ShapeDtypeStruct((N, 1), jnp.float32),
    )(coeffs, W0, b0.reshape(1, D), a0.reshape(1, 1),
      W1, a1.reshape(1, 1), W2, a2.reshape(1, 1),
      W3.reshape(D, 1), b3.reshape(1, 1))


def _sc_body(scaled_hbm, row_hbm, col_hbm, out_hbm, tab_v, row_v, col_v, acc_v):
    wid = lax.axis_index("s") * NC + lax.axis_index("c")
    base = wid * EPW
    pltpu.sync_copy(scaled_hbm, tab_v)
    pltpu.sync_copy(row_hbm.at[pl.ds(base, EPW)], row_v)
    pltpu.sync_copy(col_hbm.at[pl.ds(base, EPW)], col_v)

    zeros = jnp.zeros((LANES,), jnp.float32)

    def zbody(i, _):
        acc_v[pl.ds(i * LANES, LANES)] = zeros
        return _

    lax.fori_loop(0, N // LANES, zbody, None)

    def body(i, _):
        c = col_v[pl.ds(i * LANES, LANES)]
        r = row_v[pl.ds(i * LANES, LANES)]
        vals = plsc.load_gather(tab_v, [c])
        plsc.addupdate_scatter(acc_v, [r], vals)
        return _

    lax.fori_loop(0, STEPS, body, None)
    pltpu.sync_copy(acc_v, out_hbm.at[wid])


def _sc_scatter(scaled_flat, row, col):
    mesh = plsc.VectorSubcoreMesh(core_axis_name="c", subcore_axis_name="s")
    k = functools.partial(
        pl.kernel,
        mesh=mesh,
        out_type=jax.ShapeDtypeStruct((NW, N), jnp.float32),
        scratch_types=[
            pltpu.VMEM((N,), jnp.float32),
            pltpu.VMEM((EPW,), jnp.int32),
            pltpu.VMEM((EPW,), jnp.int32),
            pltpu.VMEM((N,), jnp.float32),
        ],
        compiler_params=pltpu.CompilerParams(needs_layout_passes=False),
    )(_sc_body)
    return k(scaled_flat, row, col)


def _combine_body(p_ref, s_ref, out_ref):
    out_ref[...] = 4.0 * s_ref[...] + jnp.sum(p_ref[...], axis=0, keepdims=True)


def _combine(partials, scaled_row):
    return pl.pallas_call(
        _combine_body,
        out_shape=jax.ShapeDtypeStruct((1, N), jnp.float32),
    )(partials, scaled_row)


def kernel(coeffs, edge_index, W0, b0, a0, W1, a1, W2, a2, W3, b3):
    scaled = _mlp(coeffs, W0, b0, a0, W1, a1, W2, a2, W3, b3)
    scaled_flat = scaled.reshape(N)
    partials = _sc_scatter(scaled_flat, edge_index[0], edge_index[1])
    out = _combine(partials, scaled.reshape(1, N))
    return out.reshape(N, 1)
